# speedup vs baseline: 1.0772x; 1.0772x over previous
import jax
import jax.numpy as jnp
from jax.experimental import pallas as pl
from jax.experimental.pallas import tpu as pltpu

BS, NQG, MAX_SEQ, HEAD = 8, 8, 4096, 128
T = 16


def _body(pos_ref, k_ref, v_ref, kc_ref, vc_ref, ok_ref, ov_ref):
    ok_ref[...] = kc_ref[...]
    ov_ref[...] = vc_ref[...]
    for t in range(T):
        idx = pos_ref[t]
        ok_ref[0, 0, pl.ds(idx, 1), :] = k_ref[0, 0, pl.ds(t, 1), :]
        ov_ref[0, 0, pl.ds(idx, 1), :] = v_ref[0, 0, pl.ds(t, 1), :]


def kernel(input_pos, k, v, k_cache, v_cache):
    spec_kv = pl.BlockSpec((1, 1, T, HEAD), lambda b, g, pos: (b, g, 0, 0))
    spec_c = pl.BlockSpec((1, 1, MAX_SEQ, HEAD), lambda b, g, pos: (b, g, 0, 0))
    gs = pltpu.PrefetchScalarGridSpec(
        num_scalar_prefetch=1,
        grid=(BS, NQG),
        in_specs=[spec_kv, spec_kv, spec_c, spec_c],
        out_specs=[spec_c, spec_c],
    )
    out = pl.pallas_call(
        _body,
        grid_spec=gs,
        out_shape=[jax.ShapeDtypeStruct((BS, NQG, MAX_SEQ, HEAD), jnp.float32)] * 2,
    )(input_pos, k, v, k_cache, v_cache)
    return (out[0], out[1])
